# baseline (speedup 1.0000x reference)
import functools

import jax
import jax.numpy as jnp
from jax import lax
from jax.experimental import pallas as pl
from jax.experimental.pallas import tpu as pltpu
from jax.experimental.pallas import tpu_sc as plsc

H = 128
NMOL = 1024
APM = 48
NA = 1 + NMOL * APM
NBND = 1 + NMOL * APM * 4
MAXNB = 6

NC, NS = 2, 16
NW = NC * NS

_GC = 128
_NBUF = 3
_ALIGN = NW * _GC

A_PAD = 49664
A6 = MAXNB * A_PAD
REV_PAD = 200704
CMB = A6 + REV_PAD
A6F = 299008
B2_PAD = 200704

_DG = (((1,), (1,)), ((), ()))


def _dot(x, w):
    return lax.dot_general(x.astype(jnp.bfloat16), w.astype(jnp.bfloat16),
                           _DG, preferred_element_type=jnp.float32)


def _sc_gather(table, idx):
    b = idx.shape[0]
    rpw = b // NW
    n_ch = rpw // _GC
    mesh = plsc.VectorSubcoreMesh(core_axis_name="c", subcore_axis_name="s")

    nch_tot = b // _GC
    assert nch_tot % NW == 0
    n_iter = nch_tot // NW

    def body(tbl, idxh, outh, idxv, rowsv, sem):
        w = lax.axis_index("s") * NC + lax.axis_index("c")

        def step(i, c):
            off = (w + i * NW) * _GC
            pltpu.sync_copy(idxh.at[pl.ds(off, _GC)], idxv)
            pltpu.async_copy(tbl.at[idxv], rowsv, sem).wait()
            pltpu.sync_copy(rowsv, outh.at[pl.ds(off, _GC)])
            return c

        lax.fori_loop(0, n_iter, step, 0)

    run = pl.kernel(
        body,
        out_type=jax.ShapeDtypeStruct((b, H), jnp.float32),
        mesh=mesh,
        scratch_types=[
            pltpu.VMEM((_GC,), jnp.int32),
            pltpu.VMEM((_GC, H), jnp.float32),
            pltpu.SemaphoreType.DMA,
        ],
    )
    return run(table, idx)


def _mm_relu(x, w, blk=512):
    m, _ = x.shape
    n = w.shape[0]

    def kern(x_ref, w_ref, o_ref):
        o_ref[...] = jnp.maximum(_dot(x_ref[...], w_ref[...]), 0.0)

    return pl.pallas_call(
        kern,
        grid=(pl.cdiv(m, blk),),
        in_specs=[
            pl.BlockSpec((blk, x.shape[1]), lambda i: (i, 0)),
            pl.BlockSpec((n, w.shape[1]), lambda i: (0, 0)),
        ],
        out_specs=pl.BlockSpec((blk, n), lambda i: (i, 0)),
        out_shape=jax.ShapeDtypeStruct((m, n), jnp.float32),
    )(x, w)


def _agg(flat, m_atom, blk=512):
    nb = A_PAD // blk

    def kern(n0, n1, n2, n3, n4, n5, *rest):
        ns = (n0[...], n1[...], n2[...], n3[...], n4[...], n5[...])
        s = ns[0] + ns[1] + ns[2] + ns[3] + ns[4] + ns[5]
        mx = jnp.maximum(
            jnp.maximum(jnp.maximum(ns[0], ns[1]), jnp.maximum(ns[2], ns[3])),
            jnp.maximum(ns[4], ns[5]))
        if len(rest) == 2:
            rest[1][...] = rest[0][...] + s * mx
        else:
            rest[0][...] = s * mx

    nspecs = [
        pl.BlockSpec((blk, H), functools.partial(lambda j, i: (j * nb + i, 0), j))
        for j in range(MAXNB)
    ]
    extra = [] if m_atom is None else [pl.BlockSpec((blk, H), lambda i: (i, 0))]
    args = (flat,) * MAXNB + (() if m_atom is None else (m_atom,))
    return pl.pallas_call(
        kern,
        grid=(pl.cdiv(NA, blk),),
        in_specs=nspecs + extra,
        out_specs=pl.BlockSpec((blk, H), lambda i: (i, 0)),
        out_shape=jax.ShapeDtypeStruct((NA, H), jnp.float32),
    )(*args)


def _bond_update(a, rev, ib, w_h, blk=512):
    roff = 0

    def kern(a_ref, r_ref, ib_ref, w_ref, o_ref):
        x = a_ref[...] - r_ref[...]
        o_ref[...] = jnp.maximum(ib_ref[...] + _dot(x, w_ref[...]), 0.0)

    return pl.pallas_call(
        kern,
        grid=(pl.cdiv(NBND, blk),),
        in_specs=[
            pl.BlockSpec((blk, H), lambda i: (i, 0)),
            pl.BlockSpec((blk, H), lambda i: (roff + i, 0)),
            pl.BlockSpec((blk, H), lambda i: (i, 0)),
            pl.BlockSpec((H, H), lambda i: (0, 0)),
        ],
        out_specs=pl.BlockSpec((blk, H), lambda i: (i, 0)),
        out_shape=jax.ShapeDtypeStruct((NBND, H), jnp.float32),
    )(a, rev, ib, w_h)


def _node(agg, m_atom, ia, lr_w, gru_bias, mb=16):
    rows = mb * APM

    def kern(g_ref, m_ref, i_ref, w_ref, b_ref, msgt_ref, h0_ref):
        cat = jnp.concatenate([g_ref[...], m_ref[...], i_ref[...]], axis=1)
        node = _dot(cat, w_ref[...])
        node3 = node.reshape(mb, APM, H)
        h0_ref[...] = jnp.max(node3, axis=1)
        msg = jnp.maximum(node3 + b_ref[...].reshape(1, 1, H), 0.0)
        msgt_ref[...] = jnp.swapaxes(msg, 0, 1)

    return pl.pallas_call(
        kern,
        grid=(NMOL // mb,),
        in_specs=[
            pl.BlockSpec((rows, H), lambda i: (i, 0)),
            pl.BlockSpec((rows, H), lambda i: (i, 0)),
            pl.BlockSpec((rows, H), lambda i: (i, 0)),
            pl.BlockSpec((H, 3 * H), lambda i: (0, 0)),
            pl.BlockSpec((1, H), lambda i: (0, 0)),
        ],
        out_specs=[
            pl.BlockSpec((APM, mb, H), lambda i: (0, i, 0)),
            pl.BlockSpec((mb, H), lambda i: (i, 0)),
        ],
        out_shape=[
            jax.ShapeDtypeStruct((APM, NMOL, H), jnp.float32),
            jax.ShapeDtypeStruct((NMOL, H), jnp.float32),
        ],
    )(agg, m_atom, ia, lr_w, gru_bias)


def _gru(msg_t, h0, wif, whf, bif, bhf, wir, whr, bir, bhr):
    def step_dir(x, h, wi, wh, bi, bh):
        gi = _dot(x, wi) + bi
        gh = _dot(h, wh) + bh
        r = jax.nn.sigmoid(gi[:, :H] + gh[:, :H])
        z = jax.nn.sigmoid(gi[:, H : 2 * H] + gh[:, H : 2 * H])
        n = jnp.tanh(gi[:, 2 * H :] + r * gh[:, 2 * H :])
        return (1.0 - z) * n + z * h

    def kern(xf_ref, xr_ref, h0_ref, wif_r, whf_r, bif_r, bhf_r,
             wir_r, whr_r, bir_r, bhr_r, of_ref, or_ref, hf_s, hr_s):
        t = pl.program_id(0)

        @pl.when(t == 0)
        def _():
            hf_s[...] = h0_ref[...]
            hr_s[...] = h0_ref[...]

        hf = step_dir(xf_ref[0], hf_s[...], wif_r[...], whf_r[...],
                      bif_r[...], bhf_r[...])
        hf_s[...] = hf
        of_ref[0] = hf
        hr = step_dir(xr_ref[0], hr_s[...], wir_r[...], whr_r[...],
                      bir_r[...], bhr_r[...])
        hr_s[...] = hr
        or_ref[0] = hr

    wspec = pl.BlockSpec((3 * H, H), lambda t: (0, 0))
    bspec = pl.BlockSpec((1, 3 * H), lambda t: (0, 0))
    return pl.pallas_call(
        kern,
        grid=(APM,),
        in_specs=[
            pl.BlockSpec((1, NMOL, H), lambda t: (t, 0, 0)),
            pl.BlockSpec((1, NMOL, H), lambda t: (APM - 1 - t, 0, 0)),
            pl.BlockSpec((NMOL, H), lambda t: (0, 0)),
            wspec, wspec, bspec, bspec, wspec, wspec, bspec, bspec,
        ],
        out_specs=[
            pl.BlockSpec((1, NMOL, H), lambda t: (t, 0, 0)),
            pl.BlockSpec((1, NMOL, H), lambda t: (APM - 1 - t, 0, 0)),
        ],
        out_shape=[
            jax.ShapeDtypeStruct((APM, NMOL, H), jnp.float32),
            jax.ShapeDtypeStruct((APM, NMOL, H), jnp.float32),
        ],
        scratch_shapes=[
            pltpu.VMEM((NMOL, H), jnp.float32),
            pltpu.VMEM((NMOL, H), jnp.float32),
        ],
    )(msg_t, msg_t, h0, wif, whf, bif, bhf, wir, whr, bir, bhr)


def _out_proj(of, orr, w_o, b_o, mb=64):
    def kern(f_ref, r_ref, w_ref, b_ref, o_ref):
        f = f_ref[...].reshape(APM * mb, H)
        r = r_ref[...].reshape(APM * mb, H)
        p = _dot(jnp.concatenate([f, r], axis=1), w_ref[...]) + b_ref[...]
        p = jnp.maximum(p, 0.0).reshape(APM, mb, H)
        o_ref[...] = jnp.mean(p, axis=0)

    return pl.pallas_call(
        kern,
        grid=(NMOL // mb,),
        in_specs=[
            pl.BlockSpec((APM, mb, H), lambda i: (0, i, 0)),
            pl.BlockSpec((APM, mb, H), lambda i: (0, i, 0)),
            pl.BlockSpec((H, 2 * H), lambda i: (0, 0)),
            pl.BlockSpec((1, H), lambda i: (0, 0)),
        ],
        out_specs=pl.BlockSpec((mb, H), lambda i: (i, 0)),
        out_shape=jax.ShapeDtypeStruct((NMOL, H), jnp.float32),
    )(of, orr, w_o, b_o)


def kernel(f_atoms, f_bonds, a2b, b2a, b2revb, a_scope, W_i_atom, W_i_bond,
           W_h_0, W_h_1, lr_W, W_o_W, W_o_b, gru_bias, W_ih_f, W_hh_f,
           b_ih_f, b_hh_f, W_ih_r, W_hh_r, b_ih_r, b_hh_r):
    del a_scope

    idx_nbr = jnp.pad(
        jnp.asarray(a2b, jnp.int32), ((0, A_PAD - NA), (0, 0))
    ).T.reshape(-1)
    idx_rev = jnp.pad(jnp.asarray(b2revb, jnp.int32), (0, REV_PAD - NBND))
    idx_nbr_f = jnp.pad(idx_nbr, (0, A6F - A6))
    idx_b2a = jnp.pad(jnp.asarray(b2a, jnp.int32), (0, B2_PAD - NBND))

    ia = _mm_relu(f_atoms, W_i_atom)
    ib = _mm_relu(f_bonds, W_i_bond)

    m_atom, m_bond = ia, ib
    for w_h in (W_h_0, W_h_1):
        nei = _sc_gather(m_bond, idx_nbr_f)
        rev = _sc_gather(m_bond, idx_rev)
        m_atom = _agg(nei, m_atom)
        a = _sc_gather(m_atom, idx_b2a)
        m_bond = _bond_update(a, rev, ib, w_h)

    fin = _sc_gather(m_bond, idx_nbr_f)
    agg = _agg(fin, None)

    b2 = lambda v: v.reshape(1, -1)
    msg_t, h0 = _node(agg[1:], m_atom[1:], ia[1:], lr_W, b2(gru_bias))
    of, orr = _gru(msg_t, h0, W_ih_f, W_hh_f, b2(b_ih_f), b2(b_hh_f),
                   W_ih_r, W_hh_r, b2(b_ih_r), b2(b_hh_r))
    return _out_proj(of, orr, W_o_W, b2(W_o_b))


# speedup vs baseline: 1.2232x; 1.2232x over previous
import functools

import jax
import jax.numpy as jnp
from jax import lax
from jax.experimental import pallas as pl
from jax.experimental.pallas import tpu as pltpu
from jax.experimental.pallas import tpu_sc as plsc

H = 128
NMOL = 1024
APM = 48
NA = 1 + NMOL * APM
NBND = 1 + NMOL * APM * 4
MAXNB = 6

NC, NS = 2, 16
NW = NC * NS

_GC = 128

A_PAD = 49664
B_PAD = 197120

_DG = (((1,), (1,)), ((), ()))


def _dot(x, w):
    return lax.dot_general(x.astype(jnp.bfloat16), w.astype(jnp.bfloat16),
                           _DG, preferred_element_type=jnp.float32)


def _sc_gather(table, idx):
    b = idx.shape[0]
    nch = b // _GC
    n_iter = -(-nch // NW)
    mesh = plsc.VectorSubcoreMesh(core_axis_name="c", subcore_axis_name="s")

    def body(tbl, idxh, outh, idxv, rowsv, sem):
        w = lax.axis_index("s") * NC + lax.axis_index("c")

        def step(i, c):
            ch = w + i * NW

            @pl.when(ch < nch)
            def _():
                off = ch * _GC
                pltpu.sync_copy(idxh.at[pl.ds(off, _GC)], idxv)
                pltpu.async_copy(tbl.at[idxv], rowsv, sem).wait()
                pltpu.sync_copy(rowsv, outh.at[pl.ds(off, _GC)])

            return c

        lax.fori_loop(0, n_iter, step, 0)

    run = pl.kernel(
        body,
        out_type=jax.ShapeDtypeStruct((b, H), jnp.float32),
        mesh=mesh,
        scratch_types=[
            pltpu.VMEM((_GC,), jnp.int32),
            pltpu.VMEM((_GC, H), jnp.float32),
            pltpu.SemaphoreType.DMA,
        ],
    )
    return run(table, idx)


def _mm_relu(x, w, blk=512):
    m, _ = x.shape
    n = w.shape[0]

    def kern(x_ref, w_ref, o_ref):
        o_ref[...] = jnp.maximum(_dot(x_ref[...], w_ref[...]), 0.0)

    return pl.pallas_call(
        kern,
        grid=(pl.cdiv(m, blk),),
        in_specs=[
            pl.BlockSpec((blk, x.shape[1]), lambda i: (i, 0)),
            pl.BlockSpec((n, w.shape[1]), lambda i: (0, 0)),
        ],
        out_specs=pl.BlockSpec((blk, n), lambda i: (i, 0)),
        out_shape=jax.ShapeDtypeStruct((m, n), jnp.float32),
    )(x, w)


def _agg_update(nei, m_atom, blk=512):
    def kern(n_ref, m_ref, o_ref):
        n = n_ref[...]
        o_ref[...] = m_ref[...] + jnp.sum(n, axis=0) * jnp.max(n, axis=0)

    return pl.pallas_call(
        kern,
        grid=(pl.cdiv(NA, blk),),
        in_specs=[
            pl.BlockSpec((MAXNB, blk, H), lambda i: (0, i, 0)),
            pl.BlockSpec((blk, H), lambda i: (i, 0)),
        ],
        out_specs=pl.BlockSpec((blk, H), lambda i: (i, 0)),
        out_shape=jax.ShapeDtypeStruct((NA, H), jnp.float32),
    )(nei, m_atom)


def _agg_only(nei, blk=512):
    def kern(n_ref, o_ref):
        n = n_ref[...]
        o_ref[...] = jnp.sum(n, axis=0) * jnp.max(n, axis=0)

    return pl.pallas_call(
        kern,
        grid=(pl.cdiv(NA, blk),),
        in_specs=[pl.BlockSpec((MAXNB, blk, H), lambda i: (0, i, 0))],
        out_specs=pl.BlockSpec((blk, H), lambda i: (i, 0)),
        out_shape=jax.ShapeDtypeStruct((NA, H), jnp.float32),
    )(nei)


def _bond_update(a, r, ib, w_h, blk=512):
    def kern(a_ref, r_ref, ib_ref, w_ref, o_ref):
        x = a_ref[...] - r_ref[...]
        o_ref[...] = jnp.maximum(ib_ref[...] + _dot(x, w_ref[...]), 0.0)

    return pl.pallas_call(
        kern,
        grid=(pl.cdiv(NBND, blk),),
        in_specs=[
            pl.BlockSpec((blk, H), lambda i: (i, 0)),
            pl.BlockSpec((blk, H), lambda i: (i, 0)),
            pl.BlockSpec((blk, H), lambda i: (i, 0)),
            pl.BlockSpec((H, H), lambda i: (0, 0)),
        ],
        out_specs=pl.BlockSpec((blk, H), lambda i: (i, 0)),
        out_shape=jax.ShapeDtypeStruct((NBND, H), jnp.float32),
    )(a, r, ib, w_h)


def _node(agg, m_atom, ia, lr_w, gru_bias, mb=16):
    rows = mb * APM

    def kern(g_ref, m_ref, i_ref, w_ref, b_ref, msgt_ref, h0_ref):
        w = w_ref[...]
        node = (
            _dot(g_ref[...], w[:, 0:H])
            + _dot(m_ref[...], w[:, H : 2 * H])
            + _dot(i_ref[...], w[:, 2 * H : 3 * H])
        )
        node3 = node.reshape(mb, APM, H)
        h0_ref[...] = jnp.max(node3, axis=1)
        msg = jnp.maximum(node3 + b_ref[...].reshape(1, 1, H), 0.0)
        msgt_ref[...] = jnp.swapaxes(msg, 0, 1)

    return pl.pallas_call(
        kern,
        grid=(NMOL // mb,),
        in_specs=[
            pl.BlockSpec((rows, H), lambda i: (i, 0)),
            pl.BlockSpec((rows, H), lambda i: (i, 0)),
            pl.BlockSpec((rows, H), lambda i: (i, 0)),
            pl.BlockSpec((H, 3 * H), lambda i: (0, 0)),
            pl.BlockSpec((1, H), lambda i: (0, 0)),
        ],
        out_specs=[
            pl.BlockSpec((APM, mb, H), lambda i: (0, i, 0)),
            pl.BlockSpec((mb, H), lambda i: (i, 0)),
        ],
        out_shape=[
            jax.ShapeDtypeStruct((APM, NMOL, H), jnp.float32),
            jax.ShapeDtypeStruct((NMOL, H), jnp.float32),
        ],
    )(agg, m_atom, ia, lr_w, gru_bias)


def _gru(msg_t, h0, wif, whf, bif, bhf, wir, whr, bir, bhr):
    def step_dir(x, h, wi, wh, bi, bh):
        gi = _dot(x, wi) + bi
        gh = _dot(h, wh) + bh
        r = jax.nn.sigmoid(gi[:, :H] + gh[:, :H])
        z = jax.nn.sigmoid(gi[:, H : 2 * H] + gh[:, H : 2 * H])
        n = jnp.tanh(gi[:, 2 * H :] + r * gh[:, 2 * H :])
        return (1.0 - z) * n + z * h

    def kern(xf_ref, xr_ref, h0_ref, wif_r, whf_r, bif_r, bhf_r,
             wir_r, whr_r, bir_r, bhr_r, of_ref, or_ref, hf_s, hr_s):
        t = pl.program_id(0)

        @pl.when(t == 0)
        def _():
            hf_s[...] = h0_ref[...]
            hr_s[...] = h0_ref[...]

        hf = step_dir(xf_ref[0], hf_s[...], wif_r[...], whf_r[...],
                      bif_r[...], bhf_r[...])
        hf_s[...] = hf
        of_ref[0] = hf
        hr = step_dir(xr_ref[0], hr_s[...], wir_r[...], whr_r[...],
                      bir_r[...], bhr_r[...])
        hr_s[...] = hr
        or_ref[0] = hr

    wspec = pl.BlockSpec((3 * H, H), lambda t: (0, 0))
    bspec = pl.BlockSpec((1, 3 * H), lambda t: (0, 0))
    return pl.pallas_call(
        kern,
        grid=(APM,),
        in_specs=[
            pl.BlockSpec((1, NMOL, H), lambda t: (t, 0, 0)),
            pl.BlockSpec((1, NMOL, H), lambda t: (APM - 1 - t, 0, 0)),
            pl.BlockSpec((NMOL, H), lambda t: (0, 0)),
            wspec, wspec, bspec, bspec, wspec, wspec, bspec, bspec,
        ],
        out_specs=[
            pl.BlockSpec((1, NMOL, H), lambda t: (t, 0, 0)),
            pl.BlockSpec((1, NMOL, H), lambda t: (APM - 1 - t, 0, 0)),
        ],
        out_shape=[
            jax.ShapeDtypeStruct((APM, NMOL, H), jnp.float32),
            jax.ShapeDtypeStruct((APM, NMOL, H), jnp.float32),
        ],
        scratch_shapes=[
            pltpu.VMEM((NMOL, H), jnp.float32),
            pltpu.VMEM((NMOL, H), jnp.float32),
        ],
    )(msg_t, msg_t, h0, wif, whf, bif, bhf, wir, whr, bir, bhr)


def _out_proj(of, orr, w_o, b_o, mb=64):
    def kern(f_ref, r_ref, w_ref, b_ref, o_ref):
        w = w_ref[...]
        f = f_ref[...].reshape(APM * mb, H)
        r = r_ref[...].reshape(APM * mb, H)
        p = _dot(f, w[:, :H]) + _dot(r, w[:, H:]) + b_ref[...]
        p = jnp.maximum(p, 0.0).reshape(APM, mb, H)
        o_ref[...] = jnp.mean(p, axis=0)

    return pl.pallas_call(
        kern,
        grid=(NMOL // mb,),
        in_specs=[
            pl.BlockSpec((APM, mb, H), lambda i: (0, i, 0)),
            pl.BlockSpec((APM, mb, H), lambda i: (0, i, 0)),
            pl.BlockSpec((H, 2 * H), lambda i: (0, 0)),
            pl.BlockSpec((1, H), lambda i: (0, 0)),
        ],
        out_specs=pl.BlockSpec((mb, H), lambda i: (i, 0)),
        out_shape=jax.ShapeDtypeStruct((NMOL, H), jnp.float32),
    )(of, orr, w_o, b_o)


def kernel(f_atoms, f_bonds, a2b, b2a, b2revb, a_scope, W_i_atom, W_i_bond,
           W_h_0, W_h_1, lr_W, W_o_W, W_o_b, gru_bias, W_ih_f, W_hh_f,
           b_ih_f, b_hh_f, W_ih_r, W_hh_r, b_ih_r, b_hh_r):
    del a_scope

    idx_nbr = jnp.pad(
        jnp.asarray(a2b, jnp.int32), ((0, A_PAD - NA), (0, 0))
    ).T.reshape(-1)
    idx_b2a = jnp.pad(jnp.asarray(b2a, jnp.int32), (0, B_PAD - NBND))
    idx_rev = jnp.pad(jnp.asarray(b2revb, jnp.int32), (0, B_PAD - NBND))

    ia = _mm_relu(f_atoms, W_i_atom)
    ib = _mm_relu(f_bonds, W_i_bond)

    m_atom, m_bond = ia, ib
    for w_h in (W_h_0, W_h_1):
        nei = _sc_gather(m_bond, idx_nbr).reshape(MAXNB, A_PAD, H)
        m_atom = _agg_update(nei, m_atom)
        a = _sc_gather(m_atom, idx_b2a)
        r = _sc_gather(m_bond, idx_rev)
        m_bond = _bond_update(a, r, ib, w_h)

    nei = _sc_gather(m_bond, idx_nbr).reshape(MAXNB, A_PAD, H)
    agg = _agg_only(nei)

    b2 = lambda v: v.reshape(1, -1)
    msg_t, h0 = _node(agg[1:], m_atom[1:], ia[1:], lr_W, b2(gru_bias))
    of, orr = _gru(msg_t, h0, W_ih_f, W_hh_f, b2(b_ih_f), b2(b_hh_f),
                   W_ih_r, W_hh_r, b2(b_ih_r), b2(b_hh_r))
    return _out_proj(of, orr, W_o_W, b2(W_o_b))


# speedup vs baseline: 1.2897x; 1.0544x over previous
import functools

import jax
import jax.numpy as jnp
from jax import lax
from jax.experimental import pallas as pl
from jax.experimental.pallas import tpu as pltpu
from jax.experimental.pallas import tpu_sc as plsc

H = 128
NMOL = 1024
APM = 48
NA = 1 + NMOL * APM
NBND = 1 + NMOL * APM * 4
MAXNB = 6

NC, NS = 2, 16
NW = NC * NS

_GC = 128

A_PAD = 49664
B_PAD = 197120

_DG = (((1,), (1,)), ((), ()))


def _dot(x, w):
    return lax.dot_general(x.astype(jnp.bfloat16), w.astype(jnp.bfloat16),
                           _DG, preferred_element_type=jnp.float32)


def _sc_gather(table, idx):
    b = idx.shape[0]
    nch = b // _GC
    assert nch % 2 == 0
    n_iter = -(-(nch // 2) // NW)
    mesh = plsc.VectorSubcoreMesh(core_axis_name="c", subcore_axis_name="s")

    def body(tbl, idxh, outh, idxv, rowsv, s0, s1):
        w = lax.axis_index("s") * NC + lax.axis_index("c")

        def step(i, c):
            ch0 = 2 * (w + i * NW)

            @pl.when(ch0 < nch)
            def _():
                off0 = ch0 * _GC
                off1 = off0 + _GC
                pltpu.sync_copy(idxh.at[pl.ds(off0, _GC)], idxv.at[0])
                d0 = pltpu.async_copy(tbl.at[idxv.at[0]], rowsv.at[0], s0)
                pltpu.sync_copy(idxh.at[pl.ds(off1, _GC)], idxv.at[1])
                d1 = pltpu.async_copy(tbl.at[idxv.at[1]], rowsv.at[1], s1)
                d0.wait()
                pltpu.sync_copy(rowsv.at[0], outh.at[pl.ds(off0, _GC)])
                d1.wait()
                pltpu.sync_copy(rowsv.at[1], outh.at[pl.ds(off1, _GC)])

            return c

        lax.fori_loop(0, n_iter, step, 0)

    run = pl.kernel(
        body,
        out_type=jax.ShapeDtypeStruct((b, H), jnp.float32),
        mesh=mesh,
        scratch_types=[
            pltpu.VMEM((2, _GC), jnp.int32),
            pltpu.VMEM((2, _GC, H), jnp.float32),
            pltpu.SemaphoreType.DMA,
            pltpu.SemaphoreType.DMA,
        ],
    )
    return run(table, idx)


def _mm_relu(x, w, blk=512):
    m, _ = x.shape
    n = w.shape[0]

    def kern(x_ref, w_ref, o_ref):
        o_ref[...] = jnp.maximum(_dot(x_ref[...], w_ref[...]), 0.0)

    return pl.pallas_call(
        kern,
        grid=(pl.cdiv(m, blk),),
        in_specs=[
            pl.BlockSpec((blk, x.shape[1]), lambda i: (i, 0)),
            pl.BlockSpec((n, w.shape[1]), lambda i: (0, 0)),
        ],
        out_specs=pl.BlockSpec((blk, n), lambda i: (i, 0)),
        out_shape=jax.ShapeDtypeStruct((m, n), jnp.float32),
    )(x, w)


def _agg_update(nei, m_atom, blk=512):
    def kern(n_ref, m_ref, o_ref):
        n = n_ref[...]
        o_ref[...] = m_ref[...] + jnp.sum(n, axis=0) * jnp.max(n, axis=0)

    return pl.pallas_call(
        kern,
        grid=(pl.cdiv(NA, blk),),
        in_specs=[
            pl.BlockSpec((MAXNB, blk, H), lambda i: (0, i, 0)),
            pl.BlockSpec((blk, H), lambda i: (i, 0)),
        ],
        out_specs=pl.BlockSpec((blk, H), lambda i: (i, 0)),
        out_shape=jax.ShapeDtypeStruct((NA, H), jnp.float32),
    )(nei, m_atom)


def _agg_only(nei, blk=512):
    def kern(n_ref, o_ref):
        n = n_ref[...]
        o_ref[...] = jnp.sum(n, axis=0) * jnp.max(n, axis=0)

    return pl.pallas_call(
        kern,
        grid=(pl.cdiv(NA, blk),),
        in_specs=[pl.BlockSpec((MAXNB, blk, H), lambda i: (0, i, 0))],
        out_specs=pl.BlockSpec((blk, H), lambda i: (i, 0)),
        out_shape=jax.ShapeDtypeStruct((NA, H), jnp.float32),
    )(nei)


def _bond_update(a, r, ib, w_h, blk=512):
    def kern(a_ref, r_ref, ib_ref, w_ref, o_ref):
        x = a_ref[...] - r_ref[...]
        o_ref[...] = jnp.maximum(ib_ref[...] + _dot(x, w_ref[...]), 0.0)

    return pl.pallas_call(
        kern,
        grid=(pl.cdiv(NBND, blk),),
        in_specs=[
            pl.BlockSpec((blk, H), lambda i: (i, 0)),
            pl.BlockSpec((blk, H), lambda i: (i, 0)),
            pl.BlockSpec((blk, H), lambda i: (i, 0)),
            pl.BlockSpec((H, H), lambda i: (0, 0)),
        ],
        out_specs=pl.BlockSpec((blk, H), lambda i: (i, 0)),
        out_shape=jax.ShapeDtypeStruct((NBND, H), jnp.float32),
    )(a, r, ib, w_h)


def _node(agg, m_atom, ia, lr_w, gru_bias, mb=16):
    rows = mb * APM

    def kern(g_ref, m_ref, i_ref, w_ref, b_ref, msgt_ref, h0_ref):
        w = w_ref[...]
        node = (
            _dot(g_ref[...], w[:, 0:H])
            + _dot(m_ref[...], w[:, H : 2 * H])
            + _dot(i_ref[...], w[:, 2 * H : 3 * H])
        )
        node3 = node.reshape(mb, APM, H)
        h0_ref[...] = jnp.max(node3, axis=1)
        msg = jnp.maximum(node3 + b_ref[...].reshape(1, 1, H), 0.0)
        msgt_ref[...] = jnp.swapaxes(msg, 0, 1)

    return pl.pallas_call(
        kern,
        grid=(NMOL // mb,),
        in_specs=[
            pl.BlockSpec((rows, H), lambda i: (i, 0)),
            pl.BlockSpec((rows, H), lambda i: (i, 0)),
            pl.BlockSpec((rows, H), lambda i: (i, 0)),
            pl.BlockSpec((H, 3 * H), lambda i: (0, 0)),
            pl.BlockSpec((1, H), lambda i: (0, 0)),
        ],
        out_specs=[
            pl.BlockSpec((APM, mb, H), lambda i: (0, i, 0)),
            pl.BlockSpec((mb, H), lambda i: (i, 0)),
        ],
        out_shape=[
            jax.ShapeDtypeStruct((APM, NMOL, H), jnp.float32),
            jax.ShapeDtypeStruct((NMOL, H), jnp.float32),
        ],
    )(agg, m_atom, ia, lr_w, gru_bias)


def _gru(msg_t, h0, wif, whf, bif, bhf, wir, whr, bir, bhr):
    def step_dir(x, h, wi, wh, bi, bh):
        gi = _dot(x, wi) + bi
        gh = _dot(h, wh) + bh
        r = jax.nn.sigmoid(gi[:, :H] + gh[:, :H])
        z = jax.nn.sigmoid(gi[:, H : 2 * H] + gh[:, H : 2 * H])
        n = jnp.tanh(gi[:, 2 * H :] + r * gh[:, 2 * H :])
        return (1.0 - z) * n + z * h

    def kern(xf_ref, xr_ref, h0_ref, wif_r, whf_r, bif_r, bhf_r,
             wir_r, whr_r, bir_r, bhr_r, of_ref, or_ref, hf_s, hr_s):
        t = pl.program_id(0)

        @pl.when(t == 0)
        def _():
            hf_s[...] = h0_ref[...]
            hr_s[...] = h0_ref[...]

        hf = step_dir(xf_ref[0], hf_s[...], wif_r[...], whf_r[...],
                      bif_r[...], bhf_r[...])
        hf_s[...] = hf
        of_ref[0] = hf
        hr = step_dir(xr_ref[0], hr_s[...], wir_r[...], whr_r[...],
                      bir_r[...], bhr_r[...])
        hr_s[...] = hr
        or_ref[0] = hr

    wspec = pl.BlockSpec((3 * H, H), lambda t: (0, 0))
    bspec = pl.BlockSpec((1, 3 * H), lambda t: (0, 0))
    return pl.pallas_call(
        kern,
        grid=(APM,),
        in_specs=[
            pl.BlockSpec((1, NMOL, H), lambda t: (t, 0, 0)),
            pl.BlockSpec((1, NMOL, H), lambda t: (APM - 1 - t, 0, 0)),
            pl.BlockSpec((NMOL, H), lambda t: (0, 0)),
            wspec, wspec, bspec, bspec, wspec, wspec, bspec, bspec,
        ],
        out_specs=[
            pl.BlockSpec((1, NMOL, H), lambda t: (t, 0, 0)),
            pl.BlockSpec((1, NMOL, H), lambda t: (APM - 1 - t, 0, 0)),
        ],
        out_shape=[
            jax.ShapeDtypeStruct((APM, NMOL, H), jnp.float32),
            jax.ShapeDtypeStruct((APM, NMOL, H), jnp.float32),
        ],
        scratch_shapes=[
            pltpu.VMEM((NMOL, H), jnp.float32),
            pltpu.VMEM((NMOL, H), jnp.float32),
        ],
    )(msg_t, msg_t, h0, wif, whf, bif, bhf, wir, whr, bir, bhr)


def _out_proj(of, orr, w_o, b_o, mb=64):
    def kern(f_ref, r_ref, w_ref, b_ref, o_ref):
        w = w_ref[...]
        f = f_ref[...].reshape(APM * mb, H)
        r = r_ref[...].reshape(APM * mb, H)
        p = _dot(f, w[:, :H]) + _dot(r, w[:, H:]) + b_ref[...]
        p = jnp.maximum(p, 0.0).reshape(APM, mb, H)
        o_ref[...] = jnp.mean(p, axis=0)

    return pl.pallas_call(
        kern,
        grid=(NMOL // mb,),
        in_specs=[
            pl.BlockSpec((APM, mb, H), lambda i: (0, i, 0)),
            pl.BlockSpec((APM, mb, H), lambda i: (0, i, 0)),
            pl.BlockSpec((H, 2 * H), lambda i: (0, 0)),
            pl.BlockSpec((1, H), lambda i: (0, 0)),
        ],
        out_specs=pl.BlockSpec((mb, H), lambda i: (i, 0)),
        out_shape=jax.ShapeDtypeStruct((NMOL, H), jnp.float32),
    )(of, orr, w_o, b_o)


def kernel(f_atoms, f_bonds, a2b, b2a, b2revb, a_scope, W_i_atom, W_i_bond,
           W_h_0, W_h_1, lr_W, W_o_W, W_o_b, gru_bias, W_ih_f, W_hh_f,
           b_ih_f, b_hh_f, W_ih_r, W_hh_r, b_ih_r, b_hh_r):
    del a_scope

    idx_nbr = jnp.pad(
        jnp.asarray(a2b, jnp.int32), ((0, A_PAD - NA), (0, 0))
    ).T.reshape(-1)
    idx_b2a = jnp.pad(jnp.asarray(b2a, jnp.int32), (0, B_PAD - NBND))
    idx_rev = jnp.pad(jnp.asarray(b2revb, jnp.int32), (0, B_PAD - NBND))

    ia = _mm_relu(f_atoms, W_i_atom)
    ib = _mm_relu(f_bonds, W_i_bond)

    m_atom, m_bond = ia, ib
    for w_h in (W_h_0, W_h_1):
        nei = _sc_gather(m_bond, idx_nbr).reshape(MAXNB, A_PAD, H)
        m_atom = _agg_update(nei, m_atom)
        a = _sc_gather(m_atom, idx_b2a)
        r = _sc_gather(m_bond, idx_rev)
        m_bond = _bond_update(a, r, ib, w_h)

    nei = _sc_gather(m_bond, idx_nbr).reshape(MAXNB, A_PAD, H)
    agg = _agg_only(nei)

    b2 = lambda v: v.reshape(1, -1)
    msg_t, h0 = _node(agg[1:], m_atom[1:], ia[1:], lr_W, b2(gru_bias))
    of, orr = _gru(msg_t, h0, W_ih_f, W_hh_f, b2(b_ih_f), b2(b_hh_f),
                   W_ih_r, W_hh_r, b2(b_ih_r), b2(b_hh_r))
    return _out_proj(of, orr, W_o_W, b2(W_o_b))


# speedup vs baseline: 1.2904x; 1.0005x over previous
import functools

import jax
import jax.numpy as jnp
from jax import lax
from jax.experimental import pallas as pl
from jax.experimental.pallas import tpu as pltpu
from jax.experimental.pallas import tpu_sc as plsc

H = 128
NMOL = 1024
APM = 48
NA = 1 + NMOL * APM
NBND = 1 + NMOL * APM * 4
MAXNB = 6

NC, NS = 2, 16
NW = NC * NS

_GC = 128

A_PAD = 49664
B_PAD = 197120

_DG = (((1,), (1,)), ((), ()))


def _dot(x, w):
    return lax.dot_general(x, w, _DG, preferred_element_type=jnp.float32)


def _sigmoid(v):
    return 0.5 + 0.5 * jnp.tanh(0.5 * v)


def _sc_gather(table, idx):
    b = idx.shape[0]
    nch = b // _GC
    assert nch % 2 == 0
    n_iter = -(-(nch // 2) // NW)
    mesh = plsc.VectorSubcoreMesh(core_axis_name="c", subcore_axis_name="s")

    def body(tbl, idxh, outh, idxv, rowsv, s0, s1):
        w = lax.axis_index("s") * NC + lax.axis_index("c")

        def step(i, c):
            ch0 = 2 * (w + i * NW)

            @pl.when(ch0 < nch)
            def _():
                off0 = ch0 * _GC
                off1 = off0 + _GC
                pltpu.sync_copy(idxh.at[pl.ds(off0, _GC)], idxv.at[0])
                d0 = pltpu.async_copy(tbl.at[idxv.at[0]], rowsv.at[0], s0)
                pltpu.sync_copy(idxh.at[pl.ds(off1, _GC)], idxv.at[1])
                d1 = pltpu.async_copy(tbl.at[idxv.at[1]], rowsv.at[1], s1)
                d0.wait()
                pltpu.sync_copy(rowsv.at[0], outh.at[pl.ds(off0, _GC)])
                d1.wait()
                pltpu.sync_copy(rowsv.at[1], outh.at[pl.ds(off1, _GC)])

            return c

        lax.fori_loop(0, n_iter, step, 0)

    run = pl.kernel(
        body,
        out_type=jax.ShapeDtypeStruct((b, H), jnp.float32),
        mesh=mesh,
        scratch_types=[
            pltpu.VMEM((2, _GC), jnp.int32),
            pltpu.VMEM((2, _GC, H), jnp.float32),
            pltpu.SemaphoreType.DMA,
            pltpu.SemaphoreType.DMA,
        ],
    )
    return run(table, idx)


def _mm_relu(x, w, blk=512):
    m, _ = x.shape
    n = w.shape[0]

    def kern(x_ref, w_ref, o_ref):
        o_ref[...] = jnp.maximum(_dot(x_ref[...], w_ref[...]), 0.0)

    return pl.pallas_call(
        kern,
        grid=(pl.cdiv(m, blk),),
        in_specs=[
            pl.BlockSpec((blk, x.shape[1]), lambda i: (i, 0)),
            pl.BlockSpec((n, w.shape[1]), lambda i: (0, 0)),
        ],
        out_specs=pl.BlockSpec((blk, n), lambda i: (i, 0)),
        out_shape=jax.ShapeDtypeStruct((m, n), jnp.float32),
    )(x, w)


def _agg_update(nei, m_atom, blk=512):
    def kern(n_ref, m_ref, o_ref):
        n = n_ref[...]
        s = ((((n[0] + n[1]) + n[2]) + n[3]) + n[4]) + n[5]
        o_ref[...] = m_ref[...] + s * jnp.max(n, axis=0)

    return pl.pallas_call(
        kern,
        grid=(pl.cdiv(NA, blk),),
        in_specs=[
            pl.BlockSpec((MAXNB, blk, H), lambda i: (0, i, 0)),
            pl.BlockSpec((blk, H), lambda i: (i, 0)),
        ],
        out_specs=pl.BlockSpec((blk, H), lambda i: (i, 0)),
        out_shape=jax.ShapeDtypeStruct((NA, H), jnp.float32),
    )(nei, m_atom)


def _agg_only(nei, blk=512):
    def kern(n_ref, o_ref):
        n = n_ref[...]
        s = ((((n[0] + n[1]) + n[2]) + n[3]) + n[4]) + n[5]
        o_ref[...] = s * jnp.max(n, axis=0)

    return pl.pallas_call(
        kern,
        grid=(pl.cdiv(NA, blk),),
        in_specs=[pl.BlockSpec((MAXNB, blk, H), lambda i: (0, i, 0))],
        out_specs=pl.BlockSpec((blk, H), lambda i: (i, 0)),
        out_shape=jax.ShapeDtypeStruct((NA, H), jnp.float32),
    )(nei)


def _bond_update(a, r, ib, w_h, blk=512):
    def kern(a_ref, r_ref, ib_ref, w_ref, o_ref):
        x = a_ref[...] - r_ref[...]
        o_ref[...] = jnp.maximum(ib_ref[...] + _dot(x, w_ref[...]), 0.0)

    return pl.pallas_call(
        kern,
        grid=(pl.cdiv(NBND, blk),),
        in_specs=[
            pl.BlockSpec((blk, H), lambda i: (i, 0)),
            pl.BlockSpec((blk, H), lambda i: (i, 0)),
            pl.BlockSpec((blk, H), lambda i: (i, 0)),
            pl.BlockSpec((H, H), lambda i: (0, 0)),
        ],
        out_specs=pl.BlockSpec((blk, H), lambda i: (i, 0)),
        out_shape=jax.ShapeDtypeStruct((NBND, H), jnp.float32),
    )(a, r, ib, w_h)


def _node(agg, m_atom, ia, lr_w, gru_bias, mb=16):
    rows = mb * APM

    def kern(g_ref, m_ref, i_ref, w_ref, b_ref, msgt_ref, h0_ref):
        w = w_ref[...]
        node = (
            _dot(g_ref[...], w[:, 0:H])
            + _dot(m_ref[...], w[:, H : 2 * H])
            + _dot(i_ref[...], w[:, 2 * H : 3 * H])
        )
        node3 = node.reshape(mb, APM, H)
        h0_ref[...] = jnp.max(node3, axis=1)
        msg = jnp.maximum(node3 + b_ref[...].reshape(1, 1, H), 0.0)
        msgt_ref[...] = jnp.swapaxes(msg, 0, 1)

    return pl.pallas_call(
        kern,
        grid=(NMOL // mb,),
        in_specs=[
            pl.BlockSpec((rows, H), lambda i: (i, 0)),
            pl.BlockSpec((rows, H), lambda i: (i, 0)),
            pl.BlockSpec((rows, H), lambda i: (i, 0)),
            pl.BlockSpec((H, 3 * H), lambda i: (0, 0)),
            pl.BlockSpec((1, H), lambda i: (0, 0)),
        ],
        out_specs=[
            pl.BlockSpec((APM, mb, H), lambda i: (0, i, 0)),
            pl.BlockSpec((mb, H), lambda i: (i, 0)),
        ],
        out_shape=[
            jax.ShapeDtypeStruct((APM, NMOL, H), jnp.float32),
            jax.ShapeDtypeStruct((NMOL, H), jnp.float32),
        ],
    )(agg, m_atom, ia, lr_w, gru_bias)


def _gru(msg_t, h0, wif, whf, bif, bhf, wir, whr, bir, bhr):
    def step_dir(x, h, wi, wh, bi, bh):
        gi = _dot(x, wi) + bi
        gh = _dot(h, wh) + bh
        r = _sigmoid(gi[:, :H] + gh[:, :H])
        z = _sigmoid(gi[:, H : 2 * H] + gh[:, H : 2 * H])
        n = jnp.tanh(gi[:, 2 * H :] + r * gh[:, 2 * H :])
        return (1.0 - z) * n + z * h

    def kern(xf_ref, xr_ref, h0_ref, wif_r, whf_r, bif_r, bhf_r,
             wir_r, whr_r, bir_r, bhr_r, of_ref, or_ref, hf_s, hr_s):
        t = pl.program_id(0)

        @pl.when(t == 0)
        def _():
            hf_s[...] = h0_ref[...]
            hr_s[...] = h0_ref[...]

        hf = step_dir(xf_ref[0], hf_s[...], wif_r[...], whf_r[...],
                      bif_r[...], bhf_r[...])
        hf_s[...] = hf
        of_ref[0] = hf
        hr = step_dir(xr_ref[0], hr_s[...], wir_r[...], whr_r[...],
                      bir_r[...], bhr_r[...])
        hr_s[...] = hr
        or_ref[0] = hr

    wspec = pl.BlockSpec((3 * H, H), lambda t: (0, 0))
    bspec = pl.BlockSpec((1, 3 * H), lambda t: (0, 0))
    return pl.pallas_call(
        kern,
        grid=(APM,),
        in_specs=[
            pl.BlockSpec((1, NMOL, H), lambda t: (t, 0, 0)),
            pl.BlockSpec((1, NMOL, H), lambda t: (APM - 1 - t, 0, 0)),
            pl.BlockSpec((NMOL, H), lambda t: (0, 0)),
            wspec, wspec, bspec, bspec, wspec, wspec, bspec, bspec,
        ],
        out_specs=[
            pl.BlockSpec((1, NMOL, H), lambda t: (t, 0, 0)),
            pl.BlockSpec((1, NMOL, H), lambda t: (APM - 1 - t, 0, 0)),
        ],
        out_shape=[
            jax.ShapeDtypeStruct((APM, NMOL, H), jnp.float32),
            jax.ShapeDtypeStruct((APM, NMOL, H), jnp.float32),
        ],
        scratch_shapes=[
            pltpu.VMEM((NMOL, H), jnp.float32),
            pltpu.VMEM((NMOL, H), jnp.float32),
        ],
    )(msg_t, msg_t, h0, wif, whf, bif, bhf, wir, whr, bir, bhr)


def _out_proj(of, orr, w_o, b_o, mb=64):
    def kern(f_ref, r_ref, w_ref, b_ref, o_ref):
        w = w_ref[...]
        f = f_ref[...].reshape(APM * mb, H)
        r = r_ref[...].reshape(APM * mb, H)
        p = _dot(f, w[:, :H]) + _dot(r, w[:, H:]) + b_ref[...]
        p = jnp.maximum(p, 0.0).reshape(APM, mb, H)
        o_ref[...] = jnp.mean(p, axis=0)

    return pl.pallas_call(
        kern,
        grid=(NMOL // mb,),
        in_specs=[
            pl.BlockSpec((APM, mb, H), lambda i: (0, i, 0)),
            pl.BlockSpec((APM, mb, H), lambda i: (0, i, 0)),
            pl.BlockSpec((H, 2 * H), lambda i: (0, 0)),
            pl.BlockSpec((1, H), lambda i: (0, 0)),
        ],
        out_specs=pl.BlockSpec((mb, H), lambda i: (i, 0)),
        out_shape=jax.ShapeDtypeStruct((NMOL, H), jnp.float32),
    )(of, orr, w_o, b_o)


def kernel(f_atoms, f_bonds, a2b, b2a, b2revb, a_scope, W_i_atom, W_i_bond,
           W_h_0, W_h_1, lr_W, W_o_W, W_o_b, gru_bias, W_ih_f, W_hh_f,
           b_ih_f, b_hh_f, W_ih_r, W_hh_r, b_ih_r, b_hh_r):
    del a_scope

    idx_nbr = jnp.pad(
        jnp.asarray(a2b, jnp.int32), ((0, A_PAD - NA), (0, 0))
    ).T.reshape(-1)
    idx_b2a = jnp.pad(jnp.asarray(b2a, jnp.int32), (0, B_PAD - NBND))
    idx_rev = jnp.pad(jnp.asarray(b2revb, jnp.int32), (0, B_PAD - NBND))

    ia = _mm_relu(f_atoms, W_i_atom)
    ib = _mm_relu(f_bonds, W_i_bond)

    m_atom, m_bond = ia, ib
    for w_h in (W_h_0, W_h_1):
        nei = _sc_gather(m_bond, idx_nbr).reshape(MAXNB, A_PAD, H)
        m_atom = _agg_update(nei, m_atom)
        a = _sc_gather(m_atom, idx_b2a)
        r = _sc_gather(m_bond, idx_rev)
        m_bond = _bond_update(a, r, ib, w_h)

    nei = _sc_gather(m_bond, idx_nbr).reshape(MAXNB, A_PAD, H)
    agg = _agg_only(nei)

    b2 = lambda v: v.reshape(1, -1)
    msg_t, h0 = _node(agg[1:], m_atom[1:], ia[1:], lr_W, b2(gru_bias))
    of, orr = _gru(msg_t, h0, W_ih_f, W_hh_f, b2(b_ih_f), b2(b_hh_f),
                   W_ih_r, W_hh_r, b2(b_ih_r), b2(b_hh_r))
    return _out_proj(of, orr, W_o_W, b2(W_o_b))
